# baseline (speedup 1.0000x reference)
import functools

import jax
import jax.numpy as jnp
from jax import lax
from jax.experimental import pallas as pl
from jax.experimental.pallas import tpu as pltpu
from jax.experimental.pallas import tpu_sc as plsc

_N = 10000
_E = 320000
_DIN = 128

_NC = 2
_NS = 16
_NW = _NC * _NS
_EPW = _E // _NW
_CH = 80
_NCHUNK = _EPW // _CH
_NP = 10240
_HALF = _NP // 2
_TRASH = _HALF
_ACCR = 5248
_ZD = _HALF // _NS
_BB = 128
_LCH = 128
_LW = _LCH * _CH


@functools.cache
def _get_mesh():
    return plsc.VectorSubcoreMesh(core_axis_name="c", subcore_axis_name="s")


def _degree_body(src_hbm, dst_hbm, os_hbm, od_hbm, osl_hbm, odl_hbm,
                 osh_hbm, odh_hbm, ocnt_hbm,
                 sv, dv, sl_v, dl_v, sh_v, dh_v, hs_v, hd_v, cnt_v):
    cid = lax.axis_index("c")
    sid = lax.axis_index("s")
    wid = sid * _NC + cid
    zero16 = jnp.zeros((16,), jnp.float32)
    ones16 = jnp.ones((16,), jnp.float32)
    zero16i = jnp.zeros((16,), jnp.int32)
    trash16 = jnp.full((16,), _TRASH, jnp.int32)
    half16 = jnp.full((16,), _HALF, jnp.int32)

    def zero_body(i, carry):
        hs_v[pl.ds(i * 16, 16)] = zero16
        hd_v[pl.ds(i * 16, 16)] = zero16
        return carry

    lax.fori_loop(0, _N // 16, zero_body, 0)

    def dummy_body(i, carry):
        sl_v[pl.ds(i * 16, 16)] = zero16i
        sh_v[pl.ds(i * 16, 16)] = zero16i
        dl_v[pl.ds(i * 16, 16)] = trash16
        dh_v[pl.ds(i * 16, 16)] = trash16
        return carry

    lax.fori_loop(0, _LW // 16, dummy_body, 0)

    pltpu.sync_copy(src_hbm.at[wid], sv)
    pltpu.sync_copy(dst_hbm.at[wid], dv)

    def acc_body(j, offs):
        off_lo, off_hi = offs
        r = j // (_CH // 16)
        c = (j % (_CH // 16)) * 16
        s16 = sv[r, pl.ds(c, 16)]
        d16 = dv[r, pl.ds(c, 16)]
        plsc.addupdate_scatter(hs_v, [s16], ones16)
        plsc.addupdate_scatter(hd_v, [d16], ones16)
        m = d16 < half16
        nm = jnp.logical_not(m)
        plsc.store_compressed(sl_v.at[pl.ds(off_lo, 16)], s16, mask=m)
        plsc.store_compressed(dl_v.at[pl.ds(off_lo, 16)], d16, mask=m)
        plsc.store_compressed(sh_v.at[pl.ds(off_hi, 16)], s16, mask=nm)
        plsc.store_compressed(dh_v.at[pl.ds(off_hi, 16)], d16 - half16, mask=nm)
        pc = jnp.sum(m.astype(jnp.int32))
        return off_lo + pc, off_hi + (16 - pc)

    off_lo, off_hi = lax.fori_loop(0, _EPW // 16, acc_body, (0, 0))

    npl = (off_lo + 2 * _CH - 1) // (2 * _CH)
    nph = (off_hi + 2 * _CH - 1) // (2 * _CH)
    cnt_v[0, pl.ds(0, 16)] = jnp.full((16,), 1, jnp.int32) * npl
    cnt_v[1, pl.ds(0, 16)] = jnp.full((16,), 1, jnp.int32) * nph

    pltpu.sync_copy(hs_v, os_hbm.at[wid])
    pltpu.sync_copy(hd_v, od_hbm.at[wid])
    pltpu.sync_copy(sl_v, osl_hbm.at[wid])
    pltpu.sync_copy(dl_v, odl_hbm.at[wid])
    pltpu.sync_copy(sh_v, osh_hbm.at[wid])
    pltpu.sync_copy(dh_v, odh_hbm.at[wid])
    pltpu.sync_copy(cnt_v, ocnt_hbm.at[wid])


@functools.cache
def _get_degree_kernel():
    return functools.partial(
        pl.kernel,
        out_type=(jax.ShapeDtypeStruct((_NW, _N), jnp.float32),
                  jax.ShapeDtypeStruct((_NW, _N), jnp.float32),
                  jax.ShapeDtypeStruct((_NW, _LW), jnp.int32),
                  jax.ShapeDtypeStruct((_NW, _LW), jnp.int32),
                  jax.ShapeDtypeStruct((_NW, _LW), jnp.int32),
                  jax.ShapeDtypeStruct((_NW, _LW), jnp.int32),
                  jax.ShapeDtypeStruct((_NW, 2, 16), jnp.int32)),
        mesh=_get_mesh(),
        compiler_params=pltpu.CompilerParams(needs_layout_passes=False),
        scratch_types=[
            pltpu.VMEM((_NCHUNK, _CH), jnp.int32),
            pltpu.VMEM((_NCHUNK, _CH), jnp.int32),
            pltpu.VMEM((_LW,), jnp.int32),
            pltpu.VMEM((_LW,), jnp.int32),
            pltpu.VMEM((_LW,), jnp.int32),
            pltpu.VMEM((_LW,), jnp.int32),
            pltpu.VMEM((_N,), jnp.float32),
            pltpu.VMEM((_N,), jnp.float32),
            pltpu.VMEM((2, 16), jnp.int32),
        ],
    )(_degree_body)


def _agg_body(h_hbm, sl_hbm, dl_hbm, sh_hbm, dh_hbm, cnt_hbm, out_hbm,
              si_v, dp_v, cv, rows0_v, rows1_v, zb_v, db_v, acc_sh, sem0, sem1):
    cid = lax.axis_index("c")
    sid = lax.axis_index("s")
    wid = sid * _NC + cid
    zero16 = jnp.zeros((16,), jnp.float32)

    def zb_row(r, carry):
        def zb_col(j, carry2):
            zb_v[r, pl.ds(j * 16, 16)] = zero16
            return carry2
        return lax.fori_loop(0, _DIN // 16, zb_col, carry)

    lax.fori_loop(0, _BB, zb_row, 0)
    pltpu.sync_copy(cnt_hbm.at[wid], cv)

    for p, (s_hbm, d_hbm) in ((0, (sl_hbm, dl_hbm)), (1, (sh_hbm, dh_hbm))):
        pltpu.sync_copy(s_hbm.at[wid], si_v)
        pltpu.sync_copy(d_hbm.at[wid], dp_v)
        npairs = cv[p, pl.ds(0, 16)][0]
        base = sid * _ZD
        for off, cnt in ((0, _BB), (_BB, _BB), (2 * _BB, _ZD - 2 * _BB)):
            pltpu.sync_copy(zb_v.at[pl.ds(0, cnt)], acc_sh.at[pl.ds(base + off, cnt)])
        plsc.subcore_barrier()

        pltpu.async_copy(h_hbm.at[si_v.at[0]], rows0_v, sem0)

        def body(i2, carry):
            i = 2 * i2
            pltpu.async_copy(h_hbm.at[si_v.at[i + 1]], rows1_v, sem1)
            pltpu.make_async_copy(h_hbm.at[si_v.at[0]], rows0_v, sem0).wait()
            pltpu.sync_copy(rows0_v, acc_sh.at[dp_v.at[i]], add=True)
            pltpu.async_copy(h_hbm.at[si_v.at[i + 2]], rows0_v, sem0)
            pltpu.make_async_copy(h_hbm.at[si_v.at[0]], rows1_v, sem1).wait()
            pltpu.sync_copy(rows1_v, acc_sh.at[dp_v.at[i + 1]], add=True)
            return carry

        lax.fori_loop(0, npairs, body, 0)
        pltpu.make_async_copy(h_hbm.at[si_v.at[0]], rows0_v, sem0).wait()
        plsc.subcore_barrier()

        for off, cnt in ((0, _BB), (_BB, _BB), (2 * _BB, _ZD - 2 * _BB)):
            pltpu.sync_copy(acc_sh.at[pl.ds(base + off, cnt)], db_v.at[pl.ds(0, cnt)])
            pltpu.sync_copy(db_v.at[pl.ds(0, cnt)],
                            out_hbm.at[cid, pl.ds(p * _HALF + base + off, cnt)])
        plsc.subcore_barrier()


@functools.cache
def _get_agg_kernel():
    return functools.partial(
        pl.kernel,
        out_type=jax.ShapeDtypeStruct((_NC, _NP, _DIN), jnp.float32),
        mesh=_get_mesh(),
        compiler_params=pltpu.CompilerParams(needs_layout_passes=False),
        scratch_types=[
            pltpu.VMEM((_LCH, _CH), jnp.int32),
            pltpu.VMEM((_LCH, _CH), jnp.int32),
            pltpu.VMEM((2, 16), jnp.int32),
            pltpu.VMEM((_CH, _DIN), jnp.float32),
            pltpu.VMEM((_CH, _DIN), jnp.float32),
            pltpu.VMEM((_BB, _DIN), jnp.float32),
            pltpu.VMEM((_BB, _DIN), jnp.float32),
            pltpu.VMEM_SHARED((_ACCR, _DIN), jnp.float32),
            pltpu.SemaphoreType.DMA,
            pltpu.SemaphoreType.DMA,
        ],
    )(_agg_body)


_RB = 2048
_GRID = (_N + _RB - 1) // _RB


def _first_body(x_ref, ds_ref, w_ref, o_ref):
    ns = lax.rsqrt(jnp.maximum(jnp.sum(ds_ref[...], axis=0), 1.0))
    o_ref[...] = jnp.dot(x_ref[...] * ns[:, None], w_ref[...],
                         preferred_element_type=jnp.float32)


def _mid_body(agg_ref, ds_ref, dd_ref, b_ref, w_ref, o_ref):
    ns = lax.rsqrt(jnp.maximum(jnp.sum(ds_ref[...], axis=0), 1.0))
    nd = lax.rsqrt(jnp.maximum(jnp.sum(dd_ref[...], axis=0), 1.0))
    a = agg_ref[0] + agg_ref[1]
    x = jnp.maximum(a * nd[:, None] + b_ref[...], 0.0)
    o_ref[...] = jnp.dot(x * ns[:, None], w_ref[...],
                         preferred_element_type=jnp.float32)


def _last_body(agg_ref, dd_ref, b_ref, o_ref):
    nd = lax.rsqrt(jnp.maximum(jnp.sum(dd_ref[...], axis=0), 1.0))
    a = agg_ref[0] + agg_ref[1]
    o_ref[...] = jnp.maximum(a * nd[:, None] + b_ref[...], 0.0)


def _deg_spec():
    return pl.BlockSpec((_NW, _RB), lambda i: (0, i))


def _tc_first(x, deg_s, w):
    dout = w.shape[1]
    return pl.pallas_call(
        _first_body,
        grid=(_GRID,),
        in_specs=[
            pl.BlockSpec((_RB, _DIN), lambda i: (i, 0)),
            _deg_spec(),
            pl.BlockSpec(w.shape, lambda i: (0, 0)),
        ],
        out_specs=pl.BlockSpec((_RB, dout), lambda i: (i, 0)),
        out_shape=jax.ShapeDtypeStruct((_N, dout), jnp.float32),
    )(x, deg_s, w)


def _tc_mid(agg, deg_s, deg_d, b2d, w):
    din = agg.shape[2]
    dout = w.shape[1]
    return pl.pallas_call(
        _mid_body,
        grid=(_GRID,),
        in_specs=[
            pl.BlockSpec((_NC, _RB, din), lambda i: (0, i, 0)),
            _deg_spec(),
            _deg_spec(),
            pl.BlockSpec((1, din), lambda i: (0, 0)),
            pl.BlockSpec(w.shape, lambda i: (0, 0)),
        ],
        out_specs=pl.BlockSpec((_RB, dout), lambda i: (i, 0)),
        out_shape=jax.ShapeDtypeStruct((_N, dout), jnp.float32),
    )(agg, deg_s, deg_d, b2d, w)


def _tc_last(agg, deg_d, b2d):
    din = agg.shape[2]
    return pl.pallas_call(
        _last_body,
        grid=(_GRID,),
        in_specs=[
            pl.BlockSpec((_NC, _RB, din), lambda i: (0, i, 0)),
            _deg_spec(),
            pl.BlockSpec((1, din), lambda i: (0, 0)),
        ],
        out_specs=pl.BlockSpec((_RB, din), lambda i: (i, 0)),
        out_shape=jax.ShapeDtypeStruct((_N, din), jnp.float32),
    )(agg, deg_d, b2d)


def kernel(features, edge_index, W1, b1, W2, b2, W3, b3, W4, b4):
    src3 = edge_index[0].reshape(_NW, _NCHUNK, _CH)
    dst3 = edge_index[1].reshape(_NW, _NCHUNK, _CH)

    deg_s, deg_d, sl, dl, sh, dh, cnt = _get_degree_kernel()(src3, dst3)
    sl3 = sl.reshape(_NW, _LCH, _CH)
    dl3 = dl.reshape(_NW, _LCH, _CH)
    sh3 = sh.reshape(_NW, _LCH, _CH)
    dh3 = dh.reshape(_NW, _LCH, _CH)
    _agg = _get_agg_kernel()

    W1p = jnp.pad(W1, ((0, 0), (0, _DIN - W1.shape[1])))
    b1p = jnp.pad(b1, (0, _DIN - b1.shape[0]))
    W2p = jnp.pad(W2, ((0, _DIN - W2.shape[0]), (0, 0)))

    h = _tc_first(features, deg_s, W1p)
    agg = _agg(h, sl3, dl3, sh3, dh3, cnt)
    h = _tc_mid(agg, deg_s, deg_d, b1p.reshape(1, -1), W2p)
    agg = _agg(h, sl3, dl3, sh3, dh3, cnt)
    h = _tc_mid(agg, deg_s, deg_d, b2.reshape(1, -1), W3)
    agg = _agg(h, sl3, dl3, sh3, dh3, cnt)
    h = _tc_mid(agg, deg_s, deg_d, b3.reshape(1, -1), W4)
    agg = _agg(h, sl3, dl3, sh3, dh3, cnt)
    f = _tc_last(agg, deg_d, b4.reshape(1, -1))
    return f


# speedup vs baseline: 1.9909x; 1.9909x over previous
import functools

import jax
import jax.numpy as jnp
from jax import lax
from jax.experimental import pallas as pl
from jax.experimental.pallas import tpu as pltpu
from jax.experimental.pallas import tpu_sc as plsc

_N = 10000
_E = 320000
_DIN = 128

_NC = 2
_NS = 16
_NW = _NC * _NS
_EPW = _E // _NW
_CH = 80
_NCHUNK = _EPW // _CH
_NP = 10240
_HALF = _NP // 2
_TRASH = _HALF
_ACCR = 5248
_ZD = _HALF // _NS
_BB = 128


@functools.cache
def _get_mesh():
    return plsc.VectorSubcoreMesh(core_axis_name="c", subcore_axis_name="s")


def _degree_body(src_hbm, dst_hbm, os_hbm, od_hbm, olo_hbm, ohi_hbm,
                 sv, dv, lo_v, hi_v, hs_v, hd_v):
    cid = lax.axis_index("c")
    sid = lax.axis_index("s")
    wid = sid * _NC + cid
    zero16 = jnp.zeros((16,), jnp.float32)
    ones16 = jnp.ones((16,), jnp.float32)
    half16 = jnp.full((16,), _HALF, jnp.int32)

    def zero_body(i, carry):
        hs_v[pl.ds(i * 16, 16)] = zero16
        hd_v[pl.ds(i * 16, 16)] = zero16
        return carry

    lax.fori_loop(0, _N // 16, zero_body, 0)

    pltpu.sync_copy(src_hbm.at[wid], sv)
    pltpu.sync_copy(dst_hbm.at[wid], dv)

    def acc_body(j, carry):
        r = j // (_CH // 16)
        c = (j % (_CH // 16)) * 16
        s16 = sv[r, pl.ds(c, 16)]
        d16 = dv[r, pl.ds(c, 16)]
        plsc.addupdate_scatter(hs_v, [s16], ones16)
        plsc.addupdate_scatter(hd_v, [d16], ones16)
        lo_v[r, pl.ds(c, 16)] = jnp.minimum(d16, half16)
        hi_v[r, pl.ds(c, 16)] = jnp.where(d16 >= half16, d16 - half16, half16)
        return carry

    lax.fori_loop(0, _EPW // 16, acc_body, 0)

    pltpu.sync_copy(hs_v, os_hbm.at[wid])
    pltpu.sync_copy(hd_v, od_hbm.at[wid])
    pltpu.sync_copy(lo_v, olo_hbm.at[wid])
    pltpu.sync_copy(hi_v, ohi_hbm.at[wid])


@functools.cache
def _get_degree_kernel():
    return functools.partial(
        pl.kernel,
        out_type=(jax.ShapeDtypeStruct((_NW, _N), jnp.float32),
                  jax.ShapeDtypeStruct((_NW, _N), jnp.float32),
                  jax.ShapeDtypeStruct((_NW, _NCHUNK, _CH), jnp.int32),
                  jax.ShapeDtypeStruct((_NW, _NCHUNK, _CH), jnp.int32)),
        mesh=_get_mesh(),
        compiler_params=pltpu.CompilerParams(needs_layout_passes=False),
        scratch_types=[
            pltpu.VMEM((_NCHUNK, _CH), jnp.int32),
            pltpu.VMEM((_NCHUNK, _CH), jnp.int32),
            pltpu.VMEM((_NCHUNK, _CH), jnp.int32),
            pltpu.VMEM((_NCHUNK, _CH), jnp.int32),
            pltpu.VMEM((_N,), jnp.float32),
            pltpu.VMEM((_N,), jnp.float32),
        ],
    )(_degree_body)


def _agg_body(h_hbm, src_hbm, dlo_hbm, dhi_hbm, out_hbm,
              si_v, dp_v, rows0_v, rows1_v, zb_v, db_v, acc_sh, sem0, sem1):
    cid = lax.axis_index("c")
    sid = lax.axis_index("s")
    wid = sid * _NC + cid
    zero16 = jnp.zeros((16,), jnp.float32)

    def zb_row(r, carry):
        def zb_col(j, carry2):
            zb_v[r, pl.ds(j * 16, 16)] = zero16
            return carry2
        return lax.fori_loop(0, _DIN // 16, zb_col, carry)

    lax.fori_loop(0, _BB, zb_row, 0)

    pltpu.sync_copy(src_hbm.at[wid], si_v)

    for p, didx_hbm in ((0, dlo_hbm), (1, dhi_hbm)):
        pltpu.sync_copy(didx_hbm.at[wid], dp_v)
        base = sid * _ZD
        for off, cnt in ((0, _BB), (_BB, _BB), (2 * _BB, _ZD - 2 * _BB)):
            pltpu.sync_copy(zb_v.at[pl.ds(0, cnt)], acc_sh.at[pl.ds(base + off, cnt)])
        plsc.subcore_barrier()

        pltpu.async_copy(h_hbm.at[si_v.at[0]], rows0_v, sem0)

        def body(i2, carry):
            i = 2 * i2
            pltpu.async_copy(h_hbm.at[si_v.at[i + 1]], rows1_v, sem1)
            pltpu.make_async_copy(h_hbm.at[si_v.at[0]], rows0_v, sem0).wait()
            pltpu.sync_copy(rows0_v, acc_sh.at[dp_v.at[i]], add=True)
            pltpu.async_copy(h_hbm.at[si_v.at[i + 2]], rows0_v, sem0)
            pltpu.make_async_copy(h_hbm.at[si_v.at[0]], rows1_v, sem1).wait()
            pltpu.sync_copy(rows1_v, acc_sh.at[dp_v.at[i + 1]], add=True)
            return carry

        lax.fori_loop(0, (_NCHUNK - 1) // 2, body, 0)
        pltpu.make_async_copy(h_hbm.at[si_v.at[0]], rows0_v, sem0).wait()
        pltpu.sync_copy(rows0_v, acc_sh.at[dp_v.at[_NCHUNK - 1]], add=True)
        plsc.subcore_barrier()

        for off, cnt in ((0, _BB), (_BB, _BB), (2 * _BB, _ZD - 2 * _BB)):
            pltpu.sync_copy(acc_sh.at[pl.ds(base + off, cnt)], db_v.at[pl.ds(0, cnt)])
            pltpu.sync_copy(db_v.at[pl.ds(0, cnt)],
                            out_hbm.at[cid, pl.ds(p * _HALF + base + off, cnt)])
        plsc.subcore_barrier()


@functools.cache
def _get_agg_kernel():
    return functools.partial(
        pl.kernel,
        out_type=jax.ShapeDtypeStruct((_NC, _NP, _DIN), jnp.float32),
        mesh=_get_mesh(),
        compiler_params=pltpu.CompilerParams(needs_layout_passes=False),
        scratch_types=[
            pltpu.VMEM((_NCHUNK, _CH), jnp.int32),
            pltpu.VMEM((_NCHUNK, _CH), jnp.int32),
            pltpu.VMEM((_CH, _DIN), jnp.float32),
            pltpu.VMEM((_CH, _DIN), jnp.float32),
            pltpu.VMEM((_BB, _DIN), jnp.float32),
            pltpu.VMEM((_BB, _DIN), jnp.float32),
            pltpu.VMEM_SHARED((_ACCR, _DIN), jnp.float32),
            pltpu.SemaphoreType.DMA,
            pltpu.SemaphoreType.DMA,
        ],
    )(_agg_body)


_RB = 2048
_GRID = (_N + _RB - 1) // _RB


def _first_body(x_ref, ds_ref, w_ref, o_ref):
    ns = lax.rsqrt(jnp.maximum(jnp.sum(ds_ref[...], axis=0), 1.0))
    o_ref[...] = jnp.dot(x_ref[...] * ns[:, None], w_ref[...],
                         preferred_element_type=jnp.float32)


def _mid_body(agg_ref, ds_ref, dd_ref, b_ref, w_ref, o_ref):
    ns = lax.rsqrt(jnp.maximum(jnp.sum(ds_ref[...], axis=0), 1.0))
    nd = lax.rsqrt(jnp.maximum(jnp.sum(dd_ref[...], axis=0), 1.0))
    a = agg_ref[0] + agg_ref[1]
    x = jnp.maximum(a * nd[:, None] + b_ref[...], 0.0)
    o_ref[...] = jnp.dot(x * ns[:, None], w_ref[...],
                         preferred_element_type=jnp.float32)


def _last_body(agg_ref, dd_ref, b_ref, o_ref):
    nd = lax.rsqrt(jnp.maximum(jnp.sum(dd_ref[...], axis=0), 1.0))
    a = agg_ref[0] + agg_ref[1]
    o_ref[...] = jnp.maximum(a * nd[:, None] + b_ref[...], 0.0)


def _deg_spec():
    return pl.BlockSpec((_NW, _RB), lambda i: (0, i))


def _tc_first(x, deg_s, w):
    dout = w.shape[1]
    return pl.pallas_call(
        _first_body,
        grid=(_GRID,),
        in_specs=[
            pl.BlockSpec((_RB, _DIN), lambda i: (i, 0)),
            _deg_spec(),
            pl.BlockSpec(w.shape, lambda i: (0, 0)),
        ],
        out_specs=pl.BlockSpec((_RB, dout), lambda i: (i, 0)),
        out_shape=jax.ShapeDtypeStruct((_N, dout), jnp.float32),
    )(x, deg_s, w)


def _tc_mid(agg, deg_s, deg_d, b2d, w):
    din = agg.shape[2]
    dout = w.shape[1]
    return pl.pallas_call(
        _mid_body,
        grid=(_GRID,),
        in_specs=[
            pl.BlockSpec((_NC, _RB, din), lambda i: (0, i, 0)),
            _deg_spec(),
            _deg_spec(),
            pl.BlockSpec((1, din), lambda i: (0, 0)),
            pl.BlockSpec(w.shape, lambda i: (0, 0)),
        ],
        out_specs=pl.BlockSpec((_RB, dout), lambda i: (i, 0)),
        out_shape=jax.ShapeDtypeStruct((_N, dout), jnp.float32),
    )(agg, deg_s, deg_d, b2d, w)


def _tc_last(agg, deg_d, b2d):
    din = agg.shape[2]
    return pl.pallas_call(
        _last_body,
        grid=(_GRID,),
        in_specs=[
            pl.BlockSpec((_NC, _RB, din), lambda i: (0, i, 0)),
            _deg_spec(),
            pl.BlockSpec((1, din), lambda i: (0, 0)),
        ],
        out_specs=pl.BlockSpec((_RB, din), lambda i: (i, 0)),
        out_shape=jax.ShapeDtypeStruct((_N, din), jnp.float32),
    )(agg, deg_d, b2d)


def kernel(features, edge_index, W1, b1, W2, b2, W3, b3, W4, b4):
    src3 = edge_index[0].reshape(_NW, _NCHUNK, _CH)
    dst3 = edge_index[1].reshape(_NW, _NCHUNK, _CH)

    deg_s, deg_d, dlo3, dhi3 = _get_degree_kernel()(src3, dst3)
    _agg = _get_agg_kernel()

    W1p = jnp.pad(W1, ((0, 0), (0, _DIN - W1.shape[1])))
    b1p = jnp.pad(b1, (0, _DIN - b1.shape[0]))
    W2p = jnp.pad(W2, ((0, _DIN - W2.shape[0]), (0, 0)))

    h = _tc_first(features, deg_s, W1p)
    agg = _agg(h, src3, dlo3, dhi3)
    h = _tc_mid(agg, deg_s, deg_d, b1p.reshape(1, -1), W2p)
    agg = _agg(h, src3, dlo3, dhi3)
    h = _tc_mid(agg, deg_s, deg_d, b2.reshape(1, -1), W3)
    agg = _agg(h, src3, dlo3, dhi3)
    h = _tc_mid(agg, deg_s, deg_d, b3.reshape(1, -1), W4)
    agg = _agg(h, src3, dlo3, dhi3)
    f = _tc_last(agg, deg_d, b4.reshape(1, -1))
    return f


# speedup vs baseline: 1.9944x; 1.0018x over previous
import functools

import jax
import jax.numpy as jnp
from jax import lax
from jax.experimental import pallas as pl
from jax.experimental.pallas import tpu as pltpu
from jax.experimental.pallas import tpu_sc as plsc

_N = 10000
_E = 320000
_DIN = 128

_NC = 2
_NS = 16
_NW = _NC * _NS
_EPW = _E // _NW
_CH = 80
_NCHUNK = _EPW // _CH
_NP = 10240
_HALF = _NP // 2
_TRASH = _HALF
_ACCR = 5248
_ZD = _HALF // _NS
_BB = 128


@functools.cache
def _get_mesh():
    return plsc.VectorSubcoreMesh(core_axis_name="c", subcore_axis_name="s")


def _degree_body(src_hbm, dst_hbm, os_hbm, od_hbm, olo_hbm, ohi_hbm,
                 sv, dv, lo_v, hi_v, hs_v, hd_v):
    cid = lax.axis_index("c")
    sid = lax.axis_index("s")
    wid = sid * _NC + cid
    zero16 = jnp.zeros((16,), jnp.float32)
    ones16 = jnp.ones((16,), jnp.float32)
    half16 = jnp.full((16,), _HALF, jnp.int32)

    def zero_body(i, carry):
        hs_v[pl.ds(i * 16, 16)] = zero16
        hd_v[pl.ds(i * 16, 16)] = zero16
        return carry

    lax.fori_loop(0, _N // 16, zero_body, 0)

    pltpu.sync_copy(src_hbm.at[wid], sv)
    pltpu.sync_copy(dst_hbm.at[wid], dv)

    def acc_body(j, carry):
        r = j // (_CH // 16)
        c = (j % (_CH // 16)) * 16
        s16 = sv[r, pl.ds(c, 16)]
        d16 = dv[r, pl.ds(c, 16)]
        plsc.addupdate_scatter(hs_v, [s16], ones16)
        plsc.addupdate_scatter(hd_v, [d16], ones16)
        lo_v[r, pl.ds(c, 16)] = jnp.minimum(d16, half16)
        hi_v[r, pl.ds(c, 16)] = jnp.where(d16 >= half16, d16 - half16, half16)
        return carry

    lax.fori_loop(0, _EPW // 16, acc_body, 0)

    pltpu.sync_copy(hs_v, os_hbm.at[wid])
    pltpu.sync_copy(hd_v, od_hbm.at[wid])
    pltpu.sync_copy(lo_v, olo_hbm.at[wid])
    pltpu.sync_copy(hi_v, ohi_hbm.at[wid])


@functools.cache
def _get_degree_kernel():
    return functools.partial(
        pl.kernel,
        out_type=(jax.ShapeDtypeStruct((_NW, _N), jnp.float32),
                  jax.ShapeDtypeStruct((_NW, _N), jnp.float32),
                  jax.ShapeDtypeStruct((_NW, _NCHUNK, _CH), jnp.int32),
                  jax.ShapeDtypeStruct((_NW, _NCHUNK, _CH), jnp.int32)),
        mesh=_get_mesh(),
        compiler_params=pltpu.CompilerParams(needs_layout_passes=False),
        scratch_types=[
            pltpu.VMEM((_NCHUNK, _CH), jnp.int32),
            pltpu.VMEM((_NCHUNK, _CH), jnp.int32),
            pltpu.VMEM((_NCHUNK, _CH), jnp.int32),
            pltpu.VMEM((_NCHUNK, _CH), jnp.int32),
            pltpu.VMEM((_N,), jnp.float32),
            pltpu.VMEM((_N,), jnp.float32),
        ],
    )(_degree_body)


def _agg_body(h_hbm, src_hbm, dlo_hbm, dhi_hbm, out_hbm,
              si_v, dp_v, rows0_v, rows1_v, zb_v, db_v, acc_sh,
              semg, sems):
    cid = lax.axis_index("c")
    sid = lax.axis_index("s")
    wid = sid * _NC + cid
    zero16 = jnp.zeros((16,), jnp.float32)

    def zb_row(r, carry):
        def zb_col(j, carry2):
            zb_v[r, pl.ds(j * 16, 16)] = zero16
            return carry2
        return lax.fori_loop(0, _DIN // 16, zb_col, carry)

    lax.fori_loop(0, _BB, zb_row, 0)

    pltpu.sync_copy(src_hbm.at[wid], si_v)

    rbuf = (rows0_v, rows1_v)
    def g_start(j, b):
        pltpu.async_copy(h_hbm.at[si_v.at[j]], rbuf[b], semg)

    def g_wait(b):
        pltpu.make_async_copy(h_hbm.at[si_v.at[0]], rbuf[b], semg).wait()

    def s_start(j, b):
        pltpu.async_copy(rbuf[b], acc_sh.at[dp_v.at[j]], sems, add=True)

    def s_wait(b):
        pltpu.make_async_copy(rbuf[b], acc_sh.at[dp_v.at[0]], sems).wait()

    def chunk(j, b, first_wait, do_issue):
        if first_wait:
            s_wait((b + 1) % 2)
        if do_issue:
            g_start(j + 1, (b + 1) % 2)
        g_wait(b)
        s_start(j, b)

    for p, didx_hbm in ((0, dlo_hbm), (1, dhi_hbm)):
        pltpu.sync_copy(didx_hbm.at[wid], dp_v)
        base = sid * _ZD
        for off, cnt in ((0, _BB), (_BB, _BB), (2 * _BB, _ZD - 2 * _BB)):
            pltpu.sync_copy(zb_v.at[pl.ds(0, cnt)], acc_sh.at[pl.ds(base + off, cnt)])
        plsc.subcore_barrier()

        g_start(0, 0)
        chunk(0, 0, False, True)

        def body(i2, carry):
            jb = 1 + 2 * i2
            chunk(jb, 1, True, True)
            chunk(jb + 1, 0, True, True)
            return carry

        lax.fori_loop(0, (_NCHUNK - 2) // 2, body, 0)
        chunk(_NCHUNK - 2, 1, True, True)
        chunk(_NCHUNK - 1, 0, True, False)
        s_wait(0)
        plsc.subcore_barrier()

        for off, cnt in ((0, _BB), (_BB, _BB), (2 * _BB, _ZD - 2 * _BB)):
            pltpu.sync_copy(acc_sh.at[pl.ds(base + off, cnt)], db_v.at[pl.ds(0, cnt)])
            pltpu.sync_copy(db_v.at[pl.ds(0, cnt)],
                            out_hbm.at[cid, pl.ds(p * _HALF + base + off, cnt)])
        plsc.subcore_barrier()


@functools.cache
def _get_agg_kernel():
    return functools.partial(
        pl.kernel,
        out_type=jax.ShapeDtypeStruct((_NC, _NP, _DIN), jnp.float32),
        mesh=_get_mesh(),
        compiler_params=pltpu.CompilerParams(needs_layout_passes=False),
        scratch_types=[
            pltpu.VMEM((_NCHUNK, _CH), jnp.int32),
            pltpu.VMEM((_NCHUNK, _CH), jnp.int32),
            pltpu.VMEM((_CH, _DIN), jnp.float32),
            pltpu.VMEM((_CH, _DIN), jnp.float32),
            pltpu.VMEM((_BB, _DIN), jnp.float32),
            pltpu.VMEM((_BB, _DIN), jnp.float32),
            pltpu.VMEM_SHARED((_ACCR, _DIN), jnp.float32),
            pltpu.SemaphoreType.DMA,
            pltpu.SemaphoreType.DMA,
        ],
    )(_agg_body)


_RB = 2048
_GRID = (_N + _RB - 1) // _RB


def _first_body(x_ref, ds_ref, w_ref, o_ref):
    ns = lax.rsqrt(jnp.maximum(jnp.sum(ds_ref[...], axis=0), 1.0))
    o_ref[...] = jnp.dot(x_ref[...] * ns[:, None], w_ref[...],
                         preferred_element_type=jnp.float32)


def _mid_body(agg_ref, ds_ref, dd_ref, b_ref, w_ref, o_ref):
    ns = lax.rsqrt(jnp.maximum(jnp.sum(ds_ref[...], axis=0), 1.0))
    nd = lax.rsqrt(jnp.maximum(jnp.sum(dd_ref[...], axis=0), 1.0))
    a = agg_ref[0] + agg_ref[1]
    x = jnp.maximum(a * nd[:, None] + b_ref[...], 0.0)
    o_ref[...] = jnp.dot(x * ns[:, None], w_ref[...],
                         preferred_element_type=jnp.float32)


def _last_body(agg_ref, dd_ref, b_ref, o_ref):
    nd = lax.rsqrt(jnp.maximum(jnp.sum(dd_ref[...], axis=0), 1.0))
    a = agg_ref[0] + agg_ref[1]
    o_ref[...] = jnp.maximum(a * nd[:, None] + b_ref[...], 0.0)


def _deg_spec():
    return pl.BlockSpec((_NW, _RB), lambda i: (0, i))


def _tc_first(x, deg_s, w):
    dout = w.shape[1]
    return pl.pallas_call(
        _first_body,
        grid=(_GRID,),
        in_specs=[
            pl.BlockSpec((_RB, _DIN), lambda i: (i, 0)),
            _deg_spec(),
            pl.BlockSpec(w.shape, lambda i: (0, 0)),
        ],
        out_specs=pl.BlockSpec((_RB, dout), lambda i: (i, 0)),
        out_shape=jax.ShapeDtypeStruct((_N, dout), jnp.float32),
    )(x, deg_s, w)


def _tc_mid(agg, deg_s, deg_d, b2d, w):
    din = agg.shape[2]
    dout = w.shape[1]
    return pl.pallas_call(
        _mid_body,
        grid=(_GRID,),
        in_specs=[
            pl.BlockSpec((_NC, _RB, din), lambda i: (0, i, 0)),
            _deg_spec(),
            _deg_spec(),
            pl.BlockSpec((1, din), lambda i: (0, 0)),
            pl.BlockSpec(w.shape, lambda i: (0, 0)),
        ],
        out_specs=pl.BlockSpec((_RB, dout), lambda i: (i, 0)),
        out_shape=jax.ShapeDtypeStruct((_N, dout), jnp.float32),
    )(agg, deg_s, deg_d, b2d, w)


def _tc_last(agg, deg_d, b2d):
    din = agg.shape[2]
    return pl.pallas_call(
        _last_body,
        grid=(_GRID,),
        in_specs=[
            pl.BlockSpec((_NC, _RB, din), lambda i: (0, i, 0)),
            _deg_spec(),
            pl.BlockSpec((1, din), lambda i: (0, 0)),
        ],
        out_specs=pl.BlockSpec((_RB, din), lambda i: (i, 0)),
        out_shape=jax.ShapeDtypeStruct((_N, din), jnp.float32),
    )(agg, deg_d, b2d)


def kernel(features, edge_index, W1, b1, W2, b2, W3, b3, W4, b4):
    src3 = edge_index[0].reshape(_NW, _NCHUNK, _CH)
    dst3 = edge_index[1].reshape(_NW, _NCHUNK, _CH)

    deg_s, deg_d, dlo3, dhi3 = _get_degree_kernel()(src3, dst3)
    _agg = _get_agg_kernel()

    W1p = jnp.pad(W1, ((0, 0), (0, _DIN - W1.shape[1])))
    b1p = jnp.pad(b1, (0, _DIN - b1.shape[0]))
    W2p = jnp.pad(W2, ((0, _DIN - W2.shape[0]), (0, 0)))

    h = _tc_first(features, deg_s, W1p)
    agg = _agg(h, src3, dlo3, dhi3)
    h = _tc_mid(agg, deg_s, deg_d, b1p.reshape(1, -1), W2p)
    agg = _agg(h, src3, dlo3, dhi3)
    h = _tc_mid(agg, deg_s, deg_d, b2.reshape(1, -1), W3)
    agg = _agg(h, src3, dlo3, dhi3)
    h = _tc_mid(agg, deg_s, deg_d, b3.reshape(1, -1), W4)
    agg = _agg(h, src3, dlo3, dhi3)
    f = _tc_last(agg, deg_d, b4.reshape(1, -1))
    return f
